# Initial kernel scaffold; baseline (speedup 1.0000x reference)
import jax
import jax.numpy as jnp
from jax.experimental import pallas as pl


def kernel(edge_index, node_ptr, edge_attr, question_emb, node_global_ids, entity_table, relation_table, W_ent, W_rel, W_query, W_edge, b_edge):
    raise NotImplementedError("write your pallas kernel here")


# speedup vs baseline: 2.1665x; 2.1665x over previous
import functools

import jax
import jax.numpy as jnp
from jax import lax
from jax.experimental import pallas as pl
from jax.experimental.pallas import tpu as pltpu
from jax.experimental.pallas import tpu_sc as plsc

D = 128
_NC, _NS = 2, 16
_NW = _NC * _NS
_HI = lax.Precision.HIGHEST


def _wid():
    return lax.axis_index("s") * _NC + lax.axis_index("c")


def _sc_mesh():
    return plsc.VectorSubcoreMesh(core_axis_name="c", subcore_axis_name="s",
                                  num_cores=_NC, num_subcores=_NS)


def _entity_gather(entity_table, ids_pad, npad):
    rows_per_w = npad // _NW
    ch = 80
    n_chunks = rows_per_w // ch

    @functools.partial(
        pl.kernel,
        out_type=jax.ShapeDtypeStruct((npad, D), jnp.float32),
        mesh=_sc_mesh(),
        scratch_types=[
            pltpu.VMEM((rows_per_w,), jnp.int32),
            pltpu.VMEM((ch, D), jnp.float32),
            pltpu.SemaphoreType.DMA,
        ],
    )
    def k(table_hbm, ids_hbm, out_hbm, idx_v, rows_v, sem):
        base = _wid() * rows_per_w
        pltpu.sync_copy(ids_hbm.at[pl.ds(base, rows_per_w)], idx_v)
        for c in range(n_chunks):
            pltpu.async_copy(
                table_hbm.at[idx_v.at[pl.ds(c * ch, ch)]], rows_v, sem).wait()
            pltpu.sync_copy(rows_v, out_hbm.at[pl.ds(base + c * ch, ch)])

    return k(entity_table, ids_pad)


def _node_proj(ent_rows, W_ent, W1, W3):
    npad = ent_rows.shape[0]
    blk = 512
    grid = npad // blk

    def body(er_ref, we_ref, w1_ref, w3_ref, nt_ref, ah_ref, at_ref):
        nt = jnp.dot(er_ref[...], we_ref[...],
                     preferred_element_type=jnp.float32, precision=_HI)
        nt_ref[...] = nt
        ah_ref[...] = jnp.dot(nt, w1_ref[...],
                              preferred_element_type=jnp.float32, precision=_HI)
        at_ref[...] = jnp.dot(nt, w3_ref[...],
                              preferred_element_type=jnp.float32, precision=_HI)

    w_spec = pl.BlockSpec((D, D), lambda i: (0, 0))
    row_spec = pl.BlockSpec((blk, D), lambda i: (i, 0))
    return pl.pallas_call(
        body,
        grid=(grid,),
        in_specs=[row_spec, w_spec, w_spec, w_spec],
        out_specs=[row_spec, row_spec, row_spec],
        out_shape=[jax.ShapeDtypeStruct((npad, D), jnp.float32)] * 3,
    )(ent_rows, W_ent, W1, W3)


def _small_proj(relation_table, W_rel, W2, b_row, question_emb, W_query):
    nrel = relation_table.shape[0]
    nb = question_emb.shape[0]

    def body(rel_ref, wr_ref, w2_ref, b_ref, q_ref, wq_ref, r_ref, qt_ref):
        rt = jnp.dot(rel_ref[...], wr_ref[...],
                     preferred_element_type=jnp.float32, precision=_HI)
        r_ref[...] = jnp.dot(rt, w2_ref[...],
                             preferred_element_type=jnp.float32,
                             precision=_HI) + b_ref[...]
        qt_ref[...] = jnp.dot(q_ref[...], wq_ref[...],
                              preferred_element_type=jnp.float32, precision=_HI)

    return pl.pallas_call(
        body,
        out_shape=[jax.ShapeDtypeStruct((nrel, D), jnp.float32),
                   jax.ShapeDtypeStruct((nb, D), jnp.float32)],
    )(relation_table, W_rel, W2, b_row, question_emb, W_query)


def _edge_batch_ptr(heads2d, node_ptr):
    nrows = heads2d.shape[0]
    nb = node_ptr.shape[0] - 1

    def body(h_ref, ptr_ref, eb_ref, ep_ref):
        h = h_ref[...]
        raw = jnp.zeros_like(h)
        kio = lax.broadcasted_iota(jnp.int32, (8, 128), 1)
        acc = jnp.zeros((8, 128), jnp.int32)
        for j in range(1, nb + 1):
            m = (h < ptr_ref[j]).astype(jnp.int32)
            raw = raw + (1 - m)
            cnt = jnp.sum(m)
            acc = acc + jnp.where(kio == j, cnt, 0)
        eb_ref[...] = jnp.minimum(raw, nb - 1)
        ep_ref[...] = acc

    return pl.pallas_call(
        body,
        in_specs=[pl.BlockSpec(memory_space=pltpu.VMEM),
                  pl.BlockSpec(memory_space=pltpu.SMEM)],
        out_shape=[jax.ShapeDtypeStruct((nrows, 128), jnp.int32),
                   jax.ShapeDtypeStruct((8, 128), jnp.int32)],
    )(heads2d, node_ptr)


def _edge_assemble(ah, at_, rtab, heads_pad, tails_pad, attr_pad, epad):
    e_per_w = epad // _NW
    CH = 128
    n_chunks = e_per_w // CH

    @functools.partial(
        pl.kernel,
        out_type=jax.ShapeDtypeStruct((epad, D), jnp.float32),
        mesh=_sc_mesh(),
        scratch_types=[
            pltpu.VMEM((e_per_w,), jnp.int32),
            pltpu.VMEM((e_per_w,), jnp.int32),
            pltpu.VMEM((e_per_w,), jnp.int32),
            pltpu.VMEM((CH, D), jnp.float32),
            pltpu.VMEM((CH, D), jnp.float32),
            pltpu.VMEM((CH, D), jnp.float32),
            pltpu.SemaphoreType.DMA,
            pltpu.SemaphoreType.DMA,
            pltpu.SemaphoreType.DMA,
        ],
    )
    def k(ah_hbm, at_hbm, r_hbm, h_hbm, t_hbm, a_hbm, out_hbm,
          ih, it, ia, bh, br, bt, sh, sr, st):
        base = _wid() * e_per_w
        pltpu.sync_copy(h_hbm.at[pl.ds(base, e_per_w)], ih)
        pltpu.sync_copy(t_hbm.at[pl.ds(base, e_per_w)], it)
        pltpu.sync_copy(a_hbm.at[pl.ds(base, e_per_w)], ia)

        def chunk(c, carry):
            off = c * CH
            dh = pltpu.async_copy(ah_hbm.at[ih.at[pl.ds(off, CH)]], bh, sh)
            dr = pltpu.async_copy(r_hbm.at[ia.at[pl.ds(off, CH)]], br, sr)
            dt = pltpu.async_copy(at_hbm.at[it.at[pl.ds(off, CH)]], bt, st)
            dh.wait()
            dr.wait()
            dt.wait()

            def row(rr, carry2):
                for j in range(D // 16):
                    sl = pl.ds(j * 16, 16)
                    bh[rr, sl] = bh[rr, sl] + br[rr, sl] + bt[rr, sl]
                return carry2

            lax.fori_loop(0, CH, row, 0)
            pltpu.sync_copy(bh, out_hbm.at[pl.ds(base + off, CH)])
            return carry

        lax.fori_loop(0, n_chunks, chunk, 0)

    return k(ah, at_, rtab, heads_pad, tails_pad, attr_pad)


def kernel(edge_index, node_ptr, edge_attr, question_emb, node_global_ids,
           entity_table, relation_table, W_ent, W_rel, W_query, W_edge, b_edge):
    N = node_global_ids.shape[0]
    E = edge_attr.shape[0]
    NPAD = ((N + 8 * _NW - 1) // (8 * _NW)) * (8 * _NW)
    EPAD = ((E + 128 * _NW - 1) // (128 * _NW)) * (128 * _NW)

    heads = edge_index[0]
    tails = edge_index[1]
    ids_pad = jnp.concatenate(
        [node_global_ids, jnp.zeros((NPAD - N,), jnp.int32)])
    heads_pad = jnp.concatenate([heads, jnp.full((EPAD - E,), N, jnp.int32)])
    tails_pad = jnp.concatenate([tails, jnp.zeros((EPAD - E,), jnp.int32)])
    attr_pad = jnp.concatenate([edge_attr, jnp.zeros((EPAD - E,), jnp.int32)])
    W1 = W_edge[0:D]
    W2 = W_edge[D:2 * D]
    W3 = W_edge[2 * D:3 * D]

    ent_rows = _entity_gather(entity_table, ids_pad, NPAD)
    nt_pad, ah, at_ = _node_proj(ent_rows, W_ent, W1, W3)
    rtab, question_tokens = _small_proj(
        relation_table, W_rel, W2, b_edge.reshape(1, D), question_emb, W_query)
    eb2d, ep_row = _edge_batch_ptr(heads_pad.reshape(EPAD // 128, 128), node_ptr)
    et_pad = _edge_assemble(ah, at_, rtab, heads_pad, tails_pad, attr_pad, EPAD)

    edge_tokens = et_pad[:E]
    node_tokens = nt_pad[:N]
    edge_batch = eb2d.reshape(EPAD)[:E]
    edge_ptr = ep_row[0, :node_ptr.shape[0]]
    return edge_tokens, node_tokens, question_tokens, edge_batch, edge_ptr
